# baseline (speedup 1.0000x reference)
import jax
import jax.numpy as jnp
from jax.experimental import pallas as pl

N_EXPERTS = 64
K_ACTIVE = 8
BT = 256
NSPLIT = 4
D_CHUNK_HINT = None


def _router_kernel(*refs):
    x_refs = refs[:NSPLIT]
    wt_ref = refs[NSPLIT]
    topi_ref, w_ref = refs[NSPLIT + 1], refs[NSPLIT + 2]

    d_chunk = x_refs[0].shape[1]
    logits = jnp.zeros((x_refs[0].shape[0], N_EXPERTS), jnp.float32)
    for s in range(NSPLIT):
        logits += jax.lax.dot_general(
            x_refs[s][...], wt_ref[pl.ds(s * d_chunk, d_chunk), :],
            dimension_numbers=(((1,), (0,)), ((), ())),
            preferred_element_type=jnp.float32,
        )

    lanes = jax.lax.broadcasted_iota(
        jnp.int32, logits.shape, 1).astype(jnp.float32)
    neg_inf = jnp.float32(-jnp.inf)
    big_lane = jnp.float32(2.0 * N_EXPERTS)

    vals = logits
    top_vs = []
    top_is = []
    for _ in range(K_ACTIVE):
        m = jnp.max(vals, axis=-1, keepdims=True)
        idx = jnp.min(jnp.where(vals == m, lanes, big_lane), axis=-1,
                      keepdims=True)
        top_vs.append(m)
        top_is.append(idx)
        vals = jnp.where(lanes == idx, neg_inf, vals)

    topv = jnp.concatenate(top_vs, axis=-1)
    topi = jnp.concatenate(top_is, axis=-1).astype(jnp.int32)

    e = jnp.exp(topv - topv[:, :1])
    w = e / jnp.sum(e, axis=-1, keepdims=True)

    topi_ref[...] = topi
    w_ref[...] = w


@jax.jit
def kernel(x, W):
    n_tokens, d_model = x.shape
    wt = W.T
    d_chunk = d_model // NSPLIT
    grid = (n_tokens // BT,)
    x_specs = [
        pl.BlockSpec((BT, d_chunk), lambda i, s=s: (i, s))
        for s in range(NSPLIT)
    ]
    topi, w = pl.pallas_call(
        _router_kernel,
        grid=grid,
        in_specs=x_specs + [
            pl.BlockSpec((d_model, N_EXPERTS), lambda i: (0, 0)),
        ],
        out_specs=[
            pl.BlockSpec((BT, K_ACTIVE), lambda i: (i, 0)),
            pl.BlockSpec((BT, K_ACTIVE), lambda i: (i, 0)),
        ],
        out_shape=[
            jax.ShapeDtypeStruct((n_tokens, K_ACTIVE), jnp.int32),
            jax.ShapeDtypeStruct((n_tokens, K_ACTIVE), jnp.float32),
        ],
    )(*([x] * NSPLIT), wt)
    return topi, w


# speedup vs baseline: 1.4173x; 1.4173x over previous
import jax
import jax.numpy as jnp
from jax.experimental import pallas as pl

N_EXPERTS = 64
K_ACTIVE = 8
BT = 1024
NSPLIT = 4
D_CHUNK_HINT = None


def _router_kernel(*refs):
    x_refs = refs[:NSPLIT]
    wt_ref = refs[NSPLIT]
    topi_ref, w_ref = refs[NSPLIT + 1], refs[NSPLIT + 2]

    d_chunk = x_refs[0].shape[1]
    logits = jnp.zeros((x_refs[0].shape[0], N_EXPERTS), jnp.float32)
    for s in range(NSPLIT):
        logits += jax.lax.dot_general(
            x_refs[s][...], wt_ref[pl.ds(s * d_chunk, d_chunk), :],
            dimension_numbers=(((1,), (0,)), ((), ())),
            preferred_element_type=jnp.float32,
        )

    lanes = jax.lax.broadcasted_iota(
        jnp.int32, logits.shape, 1).astype(jnp.float32)
    neg_inf = jnp.float32(-jnp.inf)
    big_lane = jnp.float32(2.0 * N_EXPERTS)

    vals = logits
    top_vs = []
    top_is = []
    for _ in range(K_ACTIVE):
        m = jnp.max(vals, axis=-1, keepdims=True)
        idx = jnp.min(jnp.where(vals == m, lanes, big_lane), axis=-1,
                      keepdims=True)
        top_vs.append(m)
        top_is.append(idx)
        vals = jnp.where(lanes == idx, neg_inf, vals)

    topv = jnp.concatenate(top_vs, axis=-1)
    topi = jnp.concatenate(top_is, axis=-1).astype(jnp.int32)

    e = jnp.exp(topv - topv[:, :1])
    w = e / jnp.sum(e, axis=-1, keepdims=True)

    topi_ref[...] = topi
    w_ref[...] = w


@jax.jit
def kernel(x, W):
    n_tokens, d_model = x.shape
    wt = W.T
    d_chunk = d_model // NSPLIT
    grid = (n_tokens // BT,)
    x_specs = [
        pl.BlockSpec((BT, d_chunk), lambda i, s=s: (i, s))
        for s in range(NSPLIT)
    ]
    topi, w = pl.pallas_call(
        _router_kernel,
        grid=grid,
        in_specs=x_specs + [
            pl.BlockSpec((d_model, N_EXPERTS), lambda i: (0, 0)),
        ],
        out_specs=[
            pl.BlockSpec((BT, K_ACTIVE), lambda i: (i, 0)),
            pl.BlockSpec((BT, K_ACTIVE), lambda i: (i, 0)),
        ],
        out_shape=[
            jax.ShapeDtypeStruct((n_tokens, K_ACTIVE), jnp.int32),
            jax.ShapeDtypeStruct((n_tokens, K_ACTIVE), jnp.float32),
        ],
    )(*([x] * NSPLIT), wt)
    return topi, w


# speedup vs baseline: 1.5640x; 1.1035x over previous
import jax
import jax.numpy as jnp
from jax.experimental import pallas as pl

N_EXPERTS = 64
K_ACTIVE = 8
BT = 1024
NSPLIT = 4
D_CHUNK_HINT = None


def _router_kernel(*refs):
    x_refs = refs[:NSPLIT]
    wt_ref = refs[NSPLIT]
    topi_ref, w_ref = refs[NSPLIT + 1], refs[NSPLIT + 2]

    d_chunk = x_refs[0].shape[1]
    logits = jnp.zeros((x_refs[0].shape[0], N_EXPERTS), jnp.float32)
    for s in range(NSPLIT):
        logits += jax.lax.dot_general(
            x_refs[s][...], wt_ref[pl.ds(s * d_chunk, d_chunk), :],
            dimension_numbers=(((1,), (0,)), ((), ())),
            preferred_element_type=jnp.float32,
        )

    b = jax.lax.bitcast_convert_type(logits, jnp.int32)
    lane_i = jax.lax.broadcasted_iota(jnp.int32, logits.shape, 1)
    lane_code = jnp.where(b >= 0, (N_EXPERTS - 1) - lane_i, lane_i)
    key = jax.lax.bitcast_convert_type(
        jnp.bitwise_or(jnp.bitwise_and(b, -N_EXPERTS), lane_code),
        jnp.float32)

    neg_inf = jnp.float32(-jnp.inf)
    kms = []
    for j in range(K_ACTIVE):
        km = jnp.max(key, axis=-1, keepdims=True)
        kms.append(km)
        if j + 1 < K_ACTIVE:
            key = jnp.where(key == km, neg_inf, key)

    kk = jnp.concatenate(kms, axis=-1)
    kb = jax.lax.bitcast_convert_type(kk, jnp.int32)
    lane6 = jnp.bitwise_and(kb, N_EXPERTS - 1)
    topi = jnp.where(kb < 0, lane6, (N_EXPERTS - 1) - lane6)
    topv = jax.lax.bitcast_convert_type(
        jnp.bitwise_and(kb, -N_EXPERTS), jnp.float32)

    e = jnp.exp(topv - topv[:, :1])
    w = e / jnp.sum(e, axis=-1, keepdims=True)

    topi_ref[...] = topi
    w_ref[...] = w


@jax.jit
def kernel(x, W):
    n_tokens, d_model = x.shape
    wt = W.T
    d_chunk = d_model // NSPLIT
    grid = (n_tokens // BT,)
    x_specs = [
        pl.BlockSpec((BT, d_chunk), lambda i, s=s: (i, s))
        for s in range(NSPLIT)
    ]
    topi, w = pl.pallas_call(
        _router_kernel,
        grid=grid,
        in_specs=x_specs + [
            pl.BlockSpec((d_model, N_EXPERTS), lambda i: (0, 0)),
        ],
        out_specs=[
            pl.BlockSpec((BT, K_ACTIVE), lambda i: (i, 0)),
            pl.BlockSpec((BT, K_ACTIVE), lambda i: (i, 0)),
        ],
        out_shape=[
            jax.ShapeDtypeStruct((n_tokens, K_ACTIVE), jnp.int32),
            jax.ShapeDtypeStruct((n_tokens, K_ACTIVE), jnp.float32),
        ],
    )(*([x] * NSPLIT), wt)
    return topi, w


# speedup vs baseline: 1.5646x; 1.0004x over previous
import jax
import jax.numpy as jnp
from jax.experimental import pallas as pl

N_EXPERTS = 64
K_ACTIVE = 8
BT = 1024
NSPLIT = 8
D_CHUNK_HINT = None


def _router_kernel(*refs):
    x_refs = refs[:NSPLIT]
    wt_ref = refs[NSPLIT]
    topi_ref, w_ref = refs[NSPLIT + 1], refs[NSPLIT + 2]

    d_chunk = x_refs[0].shape[1]
    logits = jnp.zeros((x_refs[0].shape[0], N_EXPERTS), jnp.float32)
    for s in range(NSPLIT):
        logits += jax.lax.dot_general(
            x_refs[s][...], wt_ref[pl.ds(s * d_chunk, d_chunk), :],
            dimension_numbers=(((1,), (0,)), ((), ())),
            preferred_element_type=jnp.float32,
        )

    b = jax.lax.bitcast_convert_type(logits, jnp.int32)
    lane_i = jax.lax.broadcasted_iota(jnp.int32, logits.shape, 1)
    lane_code = jnp.where(b >= 0, (N_EXPERTS - 1) - lane_i, lane_i)
    key = jax.lax.bitcast_convert_type(
        jnp.bitwise_or(jnp.bitwise_and(b, -N_EXPERTS), lane_code),
        jnp.float32)

    neg_inf = jnp.float32(-jnp.inf)
    kms = []
    for j in range(K_ACTIVE):
        km = jnp.max(key, axis=-1, keepdims=True)
        kms.append(km)
        if j + 1 < K_ACTIVE:
            key = jnp.where(key == km, neg_inf, key)

    kk = jnp.concatenate(kms, axis=-1)
    kb = jax.lax.bitcast_convert_type(kk, jnp.int32)
    lane6 = jnp.bitwise_and(kb, N_EXPERTS - 1)
    topi = jnp.where(kb < 0, lane6, (N_EXPERTS - 1) - lane6)
    topv = jax.lax.bitcast_convert_type(
        jnp.bitwise_and(kb, -N_EXPERTS), jnp.float32)

    e = jnp.exp(topv - topv[:, :1])
    w = e / jnp.sum(e, axis=-1, keepdims=True)

    topi_ref[...] = topi
    w_ref[...] = w


@jax.jit
def kernel(x, W):
    n_tokens, d_model = x.shape
    wt = W.T
    d_chunk = d_model // NSPLIT
    grid = (n_tokens // BT,)
    x_specs = [
        pl.BlockSpec((BT, d_chunk), lambda i, s=s: (i, s))
        for s in range(NSPLIT)
    ]
    topi, w = pl.pallas_call(
        _router_kernel,
        grid=grid,
        in_specs=x_specs + [
            pl.BlockSpec((d_model, N_EXPERTS), lambda i: (0, 0)),
        ],
        out_specs=[
            pl.BlockSpec((BT, K_ACTIVE), lambda i: (i, 0)),
            pl.BlockSpec((BT, K_ACTIVE), lambda i: (i, 0)),
        ],
        out_shape=[
            jax.ShapeDtypeStruct((n_tokens, K_ACTIVE), jnp.int32),
            jax.ShapeDtypeStruct((n_tokens, K_ACTIVE), jnp.float32),
        ],
    )(*([x] * NSPLIT), wt)
    return topi, w


# speedup vs baseline: 1.7439x; 1.1146x over previous
import jax
import jax.numpy as jnp
from jax.experimental import pallas as pl

N_EXPERTS = 64
K_ACTIVE = 8
BT = 1024
NSPLIT = 4


def _router_kernel(*refs):
    x_refs = refs[:NSPLIT]
    w_ref = refs[NSPLIT]
    topi_ref, w_out_ref = refs[NSPLIT + 1], refs[NSPLIT + 2]

    d_chunk = x_refs[0].shape[1]
    logits_t = jnp.zeros((N_EXPERTS, x_refs[0].shape[0]), jnp.float32)
    for s in range(NSPLIT):
        logits_t += jax.lax.dot_general(
            w_ref[:, pl.ds(s * d_chunk, d_chunk)], x_refs[s][...],
            dimension_numbers=(((1,), (1,)), ((), ())),
            preferred_element_type=jnp.float32,
        )

    b = jax.lax.bitcast_convert_type(logits_t, jnp.int32)
    exp_i = jax.lax.broadcasted_iota(jnp.int32, logits_t.shape, 0)
    exp_code = jnp.where(b >= 0, (N_EXPERTS - 1) - exp_i, exp_i)
    key = jax.lax.bitcast_convert_type(
        jnp.bitwise_or(jnp.bitwise_and(b, -N_EXPERTS), exp_code),
        jnp.float32)

    neg_inf = jnp.float32(-jnp.inf)
    kms = []
    for j in range(K_ACTIVE):
        km = jnp.max(key, axis=0, keepdims=True)
        kms.append(km)
        if j + 1 < K_ACTIVE:
            key = jnp.where(key == km, neg_inf, key)

    kk = jnp.concatenate(kms, axis=0)
    kb = jax.lax.bitcast_convert_type(kk, jnp.int32)
    id6 = jnp.bitwise_and(kb, N_EXPERTS - 1)
    topi_t = jnp.where(kb < 0, id6, (N_EXPERTS - 1) - id6)
    topv_t = jax.lax.bitcast_convert_type(
        jnp.bitwise_and(kb, -N_EXPERTS), jnp.float32)

    e = jnp.exp(topv_t - topv_t[:1, :])
    w_t = e / jnp.sum(e, axis=0, keepdims=True)

    topi_ref[...] = topi_t.T
    w_out_ref[...] = w_t.T


@jax.jit
def kernel(x, W):
    n_tokens, d_model = x.shape
    d_chunk = d_model // NSPLIT
    grid = (n_tokens // BT,)
    x_specs = [
        pl.BlockSpec((BT, d_chunk), lambda i, s=s: (i, s))
        for s in range(NSPLIT)
    ]
    topi, w = pl.pallas_call(
        _router_kernel,
        grid=grid,
        in_specs=x_specs + [
            pl.BlockSpec((N_EXPERTS, d_model), lambda i: (0, 0)),
        ],
        out_specs=[
            pl.BlockSpec((BT, K_ACTIVE), lambda i: (i, 0)),
            pl.BlockSpec((BT, K_ACTIVE), lambda i: (i, 0)),
        ],
        out_shape=[
            jax.ShapeDtypeStruct((n_tokens, K_ACTIVE), jnp.int32),
            jax.ShapeDtypeStruct((n_tokens, K_ACTIVE), jnp.float32),
        ],
    )(*([x] * NSPLIT), W)
    return topi, w


# speedup vs baseline: 1.7448x; 1.0005x over previous
import jax
import jax.numpy as jnp
from jax.experimental import pallas as pl

N_EXPERTS = 64
K_ACTIVE = 8
BT = 1024
NSPLIT = 1


def _router_kernel(*refs):
    x_refs = refs[:NSPLIT]
    w_ref = refs[NSPLIT]
    topi_ref, w_out_ref = refs[NSPLIT + 1], refs[NSPLIT + 2]

    d_chunk = x_refs[0].shape[1]
    logits_t = jnp.zeros((N_EXPERTS, x_refs[0].shape[0]), jnp.float32)
    for s in range(NSPLIT):
        logits_t += jax.lax.dot_general(
            w_ref[:, pl.ds(s * d_chunk, d_chunk)], x_refs[s][...],
            dimension_numbers=(((1,), (1,)), ((), ())),
            preferred_element_type=jnp.float32,
        )

    b = jax.lax.bitcast_convert_type(logits_t, jnp.int32)
    exp_i = jax.lax.broadcasted_iota(jnp.int32, logits_t.shape, 0)
    exp_code = jnp.where(b >= 0, (N_EXPERTS - 1) - exp_i, exp_i)
    key = jax.lax.bitcast_convert_type(
        jnp.bitwise_or(jnp.bitwise_and(b, -N_EXPERTS), exp_code),
        jnp.float32)

    neg_inf = jnp.float32(-jnp.inf)
    kms = []
    for j in range(K_ACTIVE):
        km = jnp.max(key, axis=0, keepdims=True)
        kms.append(km)
        if j + 1 < K_ACTIVE:
            key = jnp.where(key == km, neg_inf, key)

    kk = jnp.concatenate(kms, axis=0)
    kb = jax.lax.bitcast_convert_type(kk, jnp.int32)
    id6 = jnp.bitwise_and(kb, N_EXPERTS - 1)
    topi_t = jnp.where(kb < 0, id6, (N_EXPERTS - 1) - id6)
    topv_t = jax.lax.bitcast_convert_type(
        jnp.bitwise_and(kb, -N_EXPERTS), jnp.float32)

    e = jnp.exp(topv_t - topv_t[:1, :])
    w_t = e / jnp.sum(e, axis=0, keepdims=True)

    topi_ref[...] = topi_t.T
    w_out_ref[...] = w_t.T


@jax.jit
def kernel(x, W):
    n_tokens, d_model = x.shape
    d_chunk = d_model // NSPLIT
    grid = (n_tokens // BT,)
    x_specs = [
        pl.BlockSpec((BT, d_chunk), lambda i, s=s: (i, s))
        for s in range(NSPLIT)
    ]
    topi, w = pl.pallas_call(
        _router_kernel,
        grid=grid,
        in_specs=x_specs + [
            pl.BlockSpec((N_EXPERTS, d_model), lambda i: (0, 0)),
        ],
        out_specs=[
            pl.BlockSpec((BT, K_ACTIVE), lambda i: (i, 0)),
            pl.BlockSpec((BT, K_ACTIVE), lambda i: (i, 0)),
        ],
        out_shape=[
            jax.ShapeDtypeStruct((n_tokens, K_ACTIVE), jnp.int32),
            jax.ShapeDtypeStruct((n_tokens, K_ACTIVE), jnp.float32),
        ],
    )(*([x] * NSPLIT), W)
    return topi, w
